# baseline (device time: 89925 ns/iter reference)
import os

import jax
import jax.numpy as jnp
from jax import lax
from jax.experimental import pallas as pl
from jax.experimental.pallas import tpu as pltpu

_ABLATE = os.environ.get('KERNEL_ABLATE', '')

N_DEV = 16
SQ = 1024
SKV = 1024
H_LOCAL = 8
DH = 128
D_MODEL = 1024
HALF = D_MODEL // 2
CHUNK = 64
SUP = 256
SCALE = 0.08838834764831843


def kernel(x, Wq, K_ext, V_ext, Wo):
    def body(x_ref, wq_ref, k_ref, v_ref, wo_ref, out_ref,
             wq_s, wo_s, k_s, v_s, ctx_s,
             sa_cw, ra_cw, sa_ccw, ra_ccw,
             sb_x, rb_x,
             sc_cw, rc_cw, sc_ccw, rc_ccw,
             semsA, semsB, semsC, load_sems):
        my = lax.axis_index("i")
        q = lax.rem(my, 4)
        z = my // 4
        right_xy = 4 * z + lax.rem(q + 1, 4)
        left_xy = 4 * z + lax.rem(q + 3, 4)
        b0 = lax.rem(z, 2)
        b1 = lax.rem(z // 2, 2)
        p1 = 4 * (z + 1 - 2 * b0) + q
        p2 = 4 * (z + 2 - 4 * b1) + q

        def exchange_start(slot, sems, s_cw, r_cw, dst_cw, s_ccw, r_ccw,
                           dst_ccw):
            r1 = pltpu.make_async_remote_copy(
                src_ref=s_cw.at[slot], dst_ref=r_cw.at[slot],
                send_sem=sems.at[0, slot], recv_sem=sems.at[1, slot],
                device_id=(dst_cw,), device_id_type=pl.DeviceIdType.MESH)
            r2 = pltpu.make_async_remote_copy(
                src_ref=s_ccw.at[slot], dst_ref=r_ccw.at[slot],
                send_sem=sems.at[2, slot], recv_sem=sems.at[3, slot],
                device_id=(dst_ccw,), device_id_type=pl.DeviceIdType.MESH)
            r1.start()
            r2.start()
            return r1, r2

        def exchange(slot, sems, s_cw, r_cw, dst_cw, s_ccw, r_ccw, dst_ccw):
            r1, r2 = exchange_start(slot, sems, s_cw, r_cw, dst_cw,
                                    s_ccw, r_ccw, dst_ccw)
            r1.wait()
            r2.wait()

        def sup_rows(j):
            return pl.ds(j * SUP, SUP)

        def run_allreduce(phalf=None):
            bf16 = jnp.bfloat16
            f32 = jnp.float32

            if phalf is None:
                def phalf(j, lo, hi):
                    return out_ref[0, sup_rows(j), lo:hi]

            sa_cw[0] = phalf(q, 0, HALF).astype(bf16)
            sa_ccw[0] = phalf(q, HALF, D_MODEL).astype(bf16)
            for s in range(3):
                slot = s % 2
                r1, r2 = exchange_start(slot, semsA, sa_cw, ra_cw, right_xy,
                                        sa_ccw, ra_ccw, left_xy)
                j_cw = lax.rem(q + 7 - s, 4)
                j_ccw = lax.rem(q + 1 + s, 4)
                p_cw = phalf(j_cw, 0, HALF)
                p_ccw = phalf(j_ccw, HALF, D_MODEL)
                r1.wait()
                r2.wait()
                v_cw = ra_cw[slot].astype(f32) + p_cw
                v_ccw = ra_ccw[slot].astype(f32) + p_ccw
                if s < 2:
                    sa_cw[(s + 1) % 2] = v_cw.astype(bf16)
                    sa_ccw[(s + 1) % 2] = v_ccw.astype(bf16)
                else:
                    out_ref[0, sup_rows(j_cw), :HALF] = v_cw
                    out_ref[0, sup_rows(j_ccw), HALF:] = v_ccw

            row0_cw = lax.rem(q + 1, 4) * SUP
            row0_ccw = lax.rem(q + 3, 4) * SUP

            o1 = b0 * 128
            o1s = (1 - b0) * 128
            o2 = o1 + b1 * CHUNK
            o2s = o1 + (1 - b1) * CHUNK

            def pairwise(slot, nrows, dst):
                rd = pltpu.make_async_remote_copy(
                    src_ref=sb_x.at[slot, :nrows], dst_ref=rb_x.at[slot, :nrows],
                    send_sem=semsB.at[0, slot], recv_sem=semsB.at[1, slot],
                    device_id=(dst,), device_id_type=pl.DeviceIdType.MESH)
                rd.start()
                rd.wait()

            def pack(slot, nrows, off):
                sb_x[slot, :nrows, :HALF] = out_ref[
                    0, pl.ds(row0_cw + off, nrows), :HALF].astype(bf16)
                sb_x[slot, :nrows, HALF:] = out_ref[
                    0, pl.ds(row0_ccw + off, nrows), HALF:].astype(bf16)

            def unpack_add(slot, nrows, off):
                out_ref[0, pl.ds(row0_cw + off, nrows), :HALF] = (
                    out_ref[0, pl.ds(row0_cw + off, nrows), :HALF]
                    + rb_x[slot, :nrows, :HALF].astype(f32))
                out_ref[0, pl.ds(row0_ccw + off, nrows), HALF:] = (
                    out_ref[0, pl.ds(row0_ccw + off, nrows), HALF:]
                    + rb_x[slot, :nrows, HALF:].astype(f32))

            def unpack_store(slot, nrows, off):
                out_ref[0, pl.ds(row0_cw + off, nrows), :HALF] = \
                    rb_x[slot, :nrows, :HALF].astype(f32)
                out_ref[0, pl.ds(row0_ccw + off, nrows), HALF:] = \
                    rb_x[slot, :nrows, HALF:].astype(f32)

            pack(0, 128, o1s)
            pairwise(0, 128, p1)
            unpack_add(0, 128, o1)
            pack(1, CHUNK, o2s)
            pairwise(1, CHUNK, p2)
            unpack_add(1, CHUNK, o2)
            pack(2, CHUNK, o2)
            pairwise(2, CHUNK, p2)
            unpack_store(2, CHUNK, o2s)
            pack(3, 128, o1)
            pairwise(3, 128, p1)
            unpack_store(3, 128, o1s)

            sc_cw[0] = out_ref[0, sup_rows(lax.rem(q + 1, 4)),
                               :HALF].astype(bf16)
            sc_ccw[0] = out_ref[0, sup_rows(lax.rem(q + 3, 4)),
                                HALF:].astype(bf16)
            for g in range(3):
                slot = g % 2
                exchange(slot, semsC, sc_cw, rc_cw, right_xy,
                         sc_ccw, rc_ccw, left_xy)
                j_cw = lax.rem(q + 4 - g, 4)
                j_ccw = lax.rem(q + g, 4)
                out_ref[0, sup_rows(j_cw), :HALF] = rc_cw[slot].astype(f32)
                out_ref[0, sup_rows(j_ccw), HALF:] = rc_ccw[slot].astype(f32)
                if g < 2:
                    sc_cw[(g + 1) % 2] = rc_cw[slot]
                    sc_ccw[(g + 1) % 2] = rc_ccw[slot]

        cp_wq = pltpu.make_async_copy(
            wq_ref.at[:, pl.ds(my * D_MODEL, D_MODEL)], wq_s, load_sems.at[0])
        cp_wo = pltpu.make_async_copy(
            wo_ref.at[pl.ds(my * D_MODEL, D_MODEL), :], wo_s, load_sems.at[1])
        cp_wq.start()
        cp_wo.start()

        def kv_copy(h):
            s = h % 2
            ck = pltpu.make_async_copy(
                k_ref.at[0, :, h, :], k_s.at[s], load_sems.at[2 + 2 * s])
            cv = pltpu.make_async_copy(
                v_ref.at[0, :, h, :], v_s.at[s], load_sems.at[3 + 2 * s])
            return ck, cv

        cp_kv = kv_copy(0)
        cp_kv[0].start()
        cp_kv[1].start()

        barrier_sem = pltpu.get_barrier_semaphore()
        for nbr in (left_xy, right_xy, p1, p2):
            pl.semaphore_signal(
                barrier_sem, inc=1,
                device_id=(nbr,), device_id_type=pl.DeviceIdType.MESH,
            )
        pl.semaphore_wait(barrier_sem, 4)

        if _ABLATE == 'nocomp':
            cp_wq.wait()
            cp_wo.wait()
            cp_kv[0].wait()
            cp_kv[1].wait()
            out_ref[0] = x_ref[0]
            run_allreduce()
            return
        cp_wq.wait()
        xb = x_ref[0].astype(jnp.bfloat16)
        wqb = wq_s[...].astype(jnp.bfloat16)
        q_all = jnp.dot(xb, wqb, preferred_element_type=jnp.float32) * SCALE

        qb = lax.broadcasted_iota(jnp.int32, (SQ, SKV), 0) // 64
        kb = lax.broadcasted_iota(jnp.int32, (SQ, SKV), 1) // 64
        mask = (qb == kb) | (kb == 0) | (lax.rem(qb + kb, 3) == 0)
        bias = jnp.where(mask, 0.0, -1e9).astype(jnp.float32)

        if _ABLATE == 'noattn':
            cp_kv[0].wait()
            cp_kv[1].wait()
            ctx_s[...] = q_all.astype(jnp.bfloat16)
        else:
            for h in range(H_LOCAL):
                ck, cv = cp_kv
                ck.wait()
                cv.wait()
                if h + 1 < H_LOCAL:
                    cp_kv = kv_copy(h + 1)
                    cp_kv[0].start()
                    cp_kv[1].start()
                q_h = q_all[:, h * DH:(h + 1) * DH].astype(jnp.bfloat16)
                k_h = k_s[h % 2].astype(jnp.bfloat16)
                v_h = v_s[h % 2].astype(jnp.bfloat16)
                scores = lax.dot_general(
                    q_h, k_h, (((1,), (1,)), ((), ())),
                    preferred_element_type=jnp.float32,
                ) + bias
                if _ABLATE == 'nosoftmax':
                    w = scores
                else:
                    e = jnp.exp(scores)
                    recip = 1.0 / jnp.sum(e, axis=1, keepdims=True)
                    w = e * recip
                ctx_h = jnp.dot(w.astype(jnp.bfloat16), v_h,
                                preferred_element_type=jnp.float32)
                ctx_s[:, h * DH:(h + 1) * DH] = ctx_h.astype(jnp.bfloat16)

        cp_wo.wait()
        wob = wo_s[...].astype(jnp.bfloat16)

        def phalf(j, lo, hi):
            return jnp.dot(ctx_s[pl.ds(j * SUP, SUP), :], wob[:, lo:hi],
                           preferred_element_type=jnp.float32)

        if _ABLATE == 'noring':
            out_ref[0] = jnp.dot(ctx_s[...], wob,
                                 preferred_element_type=jnp.float32)
        else:
            run_allreduce(phalf)

    return pl.pallas_call(
        body,
        out_shape=jax.ShapeDtypeStruct((1, SQ, D_MODEL), jnp.float32),
        in_specs=[
            pl.BlockSpec(memory_space=pltpu.VMEM),
            pl.BlockSpec(memory_space=pl.ANY),
            pl.BlockSpec(memory_space=pl.ANY),
            pl.BlockSpec(memory_space=pl.ANY),
            pl.BlockSpec(memory_space=pl.ANY),
        ],
        out_specs=pl.BlockSpec(memory_space=pltpu.VMEM),
        scratch_shapes=[
            pltpu.VMEM((D_MODEL, D_MODEL), jnp.float32),
            pltpu.VMEM((D_MODEL, D_MODEL), jnp.float32),
            pltpu.VMEM((2, SKV, DH), jnp.float32),
            pltpu.VMEM((2, SKV, DH), jnp.float32),
            pltpu.VMEM((SQ, D_MODEL), jnp.bfloat16),
            pltpu.VMEM((2, SUP, HALF), jnp.bfloat16),
            pltpu.VMEM((2, SUP, HALF), jnp.bfloat16),
            pltpu.VMEM((2, SUP, HALF), jnp.bfloat16),
            pltpu.VMEM((2, SUP, HALF), jnp.bfloat16),
            pltpu.VMEM((4, 128, D_MODEL), jnp.bfloat16),
            pltpu.VMEM((4, 128, D_MODEL), jnp.bfloat16),
            pltpu.VMEM((2, SUP, HALF), jnp.bfloat16),
            pltpu.VMEM((2, SUP, HALF), jnp.bfloat16),
            pltpu.VMEM((2, SUP, HALF), jnp.bfloat16),
            pltpu.VMEM((2, SUP, HALF), jnp.bfloat16),
            pltpu.SemaphoreType.DMA((4, 2)),
            pltpu.SemaphoreType.DMA((2, 4)),
            pltpu.SemaphoreType.DMA((4, 2)),
            pltpu.SemaphoreType.DMA((6,)),
        ],
        compiler_params=pltpu.CompilerParams(collective_id=0),
    )(x, Wq, K_ext, V_ext, Wo)
